# Initial kernel scaffold; baseline (speedup 1.0000x reference)
import jax
import jax.numpy as jnp
from jax import lax
from jax.experimental import pallas as pl
from jax.experimental.pallas import tpu as pltpu
from jax.experimental.pallas import tpu_sc as plsc

VOCAB = 400001
DIM = 100
BATCH = 4096
HIST = 200

NC, NS, LANES = 2, 16, 16
NW = NC * NS

B = BATCH * HIST
CHUNK = 128
ROWS = B // CHUNK
ROWS_PER_W = ROWS // NW


def _make_kernel():
    mesh = plsc.VectorSubcoreMesh(core_axis_name="c", subcore_axis_name="s")

    @pl.kernel(
        out_type=jax.ShapeDtypeStruct((B, DIM), jnp.float32),
        mesh=mesh,
        scratch_types=[
            pltpu.VMEM((ROWS_PER_W, CHUNK), jnp.int32),
            pltpu.VMEM((CHUNK, DIM), jnp.float32),
            pltpu.SemaphoreType.DMA,
        ],
    )
    def emb_kernel(x_hbm, table_hbm, out_hbm, idx_v, rows_v, sem):
        wid = lax.axis_index("s") * NC + lax.axis_index("c")
        row0 = wid * ROWS_PER_W
        pltpu.sync_copy(x_hbm.at[pl.ds(row0, ROWS_PER_W)], idx_v)

        def body(j, carry):
            pltpu.async_copy(table_hbm.at[idx_v.at[j]], rows_v, sem).wait()
            base = (row0 + j) * CHUNK
            pltpu.sync_copy(rows_v, out_hbm.at[pl.ds(base, CHUNK)])
            return carry

        lax.fori_loop(0, ROWS_PER_W, body, 0)

    return emb_kernel


_emb = _make_kernel()


@jax.jit
def kernel(x, table):
    x2 = x.reshape(ROWS, CHUNK).astype(jnp.int32)
    out = _emb(x2, table)
    return out.reshape(BATCH, HIST, DIM)


# speedup vs baseline: 1.0708x; 1.0708x over previous
import jax
import jax.numpy as jnp
from jax import lax
from jax.experimental import pallas as pl
from jax.experimental.pallas import tpu as pltpu
from jax.experimental.pallas import tpu_sc as plsc

VOCAB = 400001
DIM = 100
BATCH = 4096
HIST = 200

VPAD = 400008
DPAD = 128

NC, NS = 2, 16
NW = NC * NS

B = BATCH * HIST
CHUNK = 128
ROWS = B // CHUNK
ROWS_PER_W = ROWS // NW
NBUF = 2
SUPER = ROWS_PER_W // NBUF


def _make_kernel():
    mesh = plsc.VectorSubcoreMesh(core_axis_name="c", subcore_axis_name="s")

    @pl.kernel(
        out_type=jax.ShapeDtypeStruct((B, DIM), jnp.float32),
        mesh=mesh,
        scratch_types=[
            pltpu.VMEM((ROWS_PER_W, CHUNK), jnp.int32),
            pltpu.VMEM((NBUF, CHUNK, DPAD), jnp.float32),
            pltpu.VMEM((NBUF, CHUNK, DIM), jnp.float32),
            pltpu.SemaphoreType.DMA((NBUF,)),
            pltpu.SemaphoreType.DMA((NBUF,)),
        ],
    )
    def emb_kernel(x_hbm, table_hbm, out_hbm, idx_v, gbuf, sbuf, gsem, wsem):
        wid = lax.axis_index("s") * NC + lax.axis_index("c")
        row0 = wid * ROWS_PER_W
        pltpu.sync_copy(x_hbm.at[pl.ds(row0, ROWS_PER_W)], idx_v)
        tail_mask = lax.iota(jnp.int32, 16) >= 12

        def gather(c, b):
            pltpu.async_copy(table_hbm.at[idx_v.at[c]], gbuf.at[b], gsem.at[b])

        def gather_wait(b):
            pltpu.make_async_copy(
                table_hbm.at[idx_v.at[0]], gbuf.at[b], gsem.at[b]
            ).wait()

        def compact(b):
            def rows4(i4, carry):
                for r in range(4):
                    i = i4 * 4 + r
                    for k in range(6):
                        sbuf[b, i, pl.ds(16 * k, 16)] = gbuf[b, i, pl.ds(16 * k, 16)]
                    tail = gbuf[b, i, pl.ds(84, 16)]
                    cur = sbuf[b, i, pl.ds(84, 16)]
                    sbuf[b, i, pl.ds(84, 16)] = jnp.where(tail_mask, tail, cur)
                return carry

            lax.fori_loop(0, CHUNK // 4, rows4, 0)

        def write(c, b):
            pltpu.async_copy(
                sbuf.at[b], out_hbm.at[pl.ds((row0 + c) * CHUNK, CHUNK)], wsem.at[b]
            )

        def write_wait(b):
            pltpu.make_async_copy(
                sbuf.at[b], out_hbm.at[pl.ds(0, CHUNK)], wsem.at[b]
            ).wait()

        for b in range(NBUF):
            gather(b, b)
        for b in range(NBUF):
            gather_wait(b)
            compact(b)
            write(b, b)
            gather(NBUF + b, b)

        def round_(g, carry):
            for b in range(NBUF):
                c = g * NBUF + b
                gather_wait(b)
                write_wait(b)
                compact(b)
                write(c, b)
                gather(c + NBUF, b)
            return carry

        lax.fori_loop(1, SUPER - 1, round_, 0)

        for b in range(NBUF):
            c = (SUPER - 1) * NBUF + b
            gather_wait(b)
            write_wait(b)
            compact(b)
            write(c, b)
        for b in range(NBUF):
            write_wait(b)

    return emb_kernel


_emb = _make_kernel()


@jax.jit
def kernel(x, table):
    x2 = x.reshape(ROWS, CHUNK).astype(jnp.int32)
    tablep = jnp.pad(table, ((0, VPAD - VOCAB), (0, DPAD - DIM)))
    out = _emb(x2, tablep)
    return out.reshape(BATCH, HIST, DIM)
